# baseline (speedup 1.0000x reference)
import jax
import jax.numpy as jnp
from jax import lax
from jax.experimental import pallas as pl
from jax.experimental.pallas import tpu as pltpu
from jax.experimental.pallas import tpu_sc as plsc

NUM_HEADS = 12
SEQ = 256
HEAD_DIM = 64
ROW_F = SEQ * HEAD_DIM
SRC_F = 2 * SEQ * HEAD_DIM
NW = 32
RPW = SEQ // NW
SEG_F = (2 * SEQ - RPW + 1) * HEAD_DIM


def _sc_body(rev_hbm, out_hbm, buf0, buf1, rsem, wsems):
    bufs = (buf0, buf1)
    c = lax.axis_index("c")
    s = lax.axis_index("s")
    w = s * 2 + c
    base_i = w * RPW
    seg0 = (SEQ - RPW + 1 - base_i) * HEAD_DIM

    def read(h, slot):
        return pltpu.make_async_copy(
            rev_hbm.at[pl.ds(h * SRC_F + seg0, (SEQ + RPW - 1) * HEAD_DIM)],
            bufs[slot],
            rsem,
        )

    def write(h, di, slot):
        r = h * SEQ + base_i + di
        return pltpu.make_async_copy(
            bufs[slot].at[pl.ds((RPW - 1 - di) * HEAD_DIM, ROW_F)],
            out_hbm.at[pl.ds(r * ROW_F, ROW_F)],
            wsems.at[slot],
        )

    read(0, 0).start()
    for h in range(NUM_HEADS):
        slot = h % 2
        read(h, slot).wait()
        if h + 1 < NUM_HEADS:
            if h >= 1:
                for di in range(RPW):
                    write(h - 1, di, (h + 1) % 2).wait()
            read(h + 1, (h + 1) % 2).start()
        for di in range(RPW):
            write(h, di, slot).start()
    for di in range(RPW):
        write(NUM_HEADS - 2, di, (NUM_HEADS - 2) % 2).wait()
    for di in range(RPW):
        write(NUM_HEADS - 1, di, (NUM_HEADS - 1) % 2).wait()


def kernel(seq_length, relative_positional_encoding):
    start = seq_length - SEQ
    sl = jax.lax.dynamic_slice(
        relative_positional_encoding,
        (0, start, 0),
        (NUM_HEADS, 2 * SEQ, HEAD_DIM),
    )
    rev = sl[:, ::-1, :].reshape(-1)

    kern = pl.kernel(
        _sc_body,
        mesh=plsc.VectorSubcoreMesh(core_axis_name="c", subcore_axis_name="s"),
        out_type=jax.ShapeDtypeStruct((NUM_HEADS * SEQ * ROW_F,), jnp.float32),
        scratch_types=[
            pltpu.VMEM(((SEQ + RPW - 1) * HEAD_DIM,), jnp.float32),
            pltpu.VMEM(((SEQ + RPW - 1) * HEAD_DIM,), jnp.float32),
            pltpu.SemaphoreType.DMA,
            pltpu.SemaphoreType.DMA((2,)),
        ],
    )
    out = kern(rev)
    return out.reshape(NUM_HEADS, SEQ, SEQ, HEAD_DIM)


# speedup vs baseline: 4.3373x; 4.3373x over previous
import jax
import jax.numpy as jnp
from jax.experimental import pallas as pl
from jax.experimental.pallas import tpu as pltpu

NUM_HEADS = 12
SEQ = 256
HEAD_DIM = 64
ROWS = 8


def _copy_kernel(revt_ref, out_ref):
    i0 = pl.program_id(0) * ROWS
    revt = revt_ref[...]
    for di in range(ROWS):
        shifted = pltpu.roll(revt, i0 + di - SEQ, axis=2)
        out_ref[:, di] = shifted[:, :, :SEQ]


def kernel(seq_length, relative_positional_encoding):
    start = seq_length - SEQ
    sl = jax.lax.dynamic_slice(
        relative_positional_encoding,
        (0, start, 0),
        (NUM_HEADS, 2 * SEQ, HEAD_DIM),
    )
    revt = sl[:, ::-1, :].transpose(0, 2, 1)

    out = pl.pallas_call(
        _copy_kernel,
        grid=(SEQ // ROWS,),
        in_specs=[
            pl.BlockSpec(
                (NUM_HEADS, HEAD_DIM, 2 * SEQ), lambda i: (0, 0, 0)
            ),
        ],
        out_specs=pl.BlockSpec(
            (NUM_HEADS, ROWS, HEAD_DIM, SEQ), lambda i: (0, i, 0, 0)
        ),
        out_shape=jax.ShapeDtypeStruct(
            (NUM_HEADS, SEQ, HEAD_DIM, SEQ), jnp.float32
        ),
    )(revt)
    return out.transpose(0, 1, 3, 2)
